# baseline (speedup 1.0000x reference)
import functools

import jax
import jax.numpy as jnp
from jax import lax
from jax.experimental import pallas as pl
from jax.experimental.pallas import tpu as pltpu
from jax.experimental.pallas import tpu_sc as plsc

_VOCAB = 1000000
_EMBED_DIM = 64
_BATCH = 16384

_INFO = plsc.get_sparse_core_info()
_NC, _NS, _L = _INFO.num_cores, _INFO.num_subcores, _INFO.num_lanes
_NW = _NC * _NS
_BPW = _BATCH // _NW
_IDXROWS = _BPW // 128
_CHUNK = 128
_NCH = _BPW // _CHUNK


def _sc_body(tgt_hbm, ctx_hbm, tabp_hbm, out_hbm,
             idx_t, idx_c, gidx_t, gidx_c, rows_t, rows_c, out_v, sem):
    wid = lax.axis_index("s") * _NC + lax.axis_index("c")
    row0 = wid * _IDXROWS

    pltpu.sync_copy(tgt_hbm.at[pl.ds(row0, _IDXROWS)], idx_t)
    pltpu.sync_copy(ctx_hbm.at[pl.ds(row0, _IDXROWS)], idx_c)

    lane = lax.iota(jnp.int32, _L)

    def chunk(ch, carry):
        for seg in range(_CHUNK // _L):
            vt = idx_t[ch, pl.ds(seg * _L, _L)]
            vc = idx_c[ch, pl.ds(seg * _L, _L)]
            gidx_t[0, pl.ds(seg * _L, _L)] = vt // 2
            gidx_c[0, pl.ds(seg * _L, _L)] = vc // 2
        ct = pltpu.async_copy(tabp_hbm.at[gidx_t.at[0]], rows_t, sem)
        cc = pltpu.async_copy(tabp_hbm.at[gidx_c.at[0]], rows_c, sem)
        ct.wait()
        cc.wait()

        for seg in range(_CHUNK // _L):
            vt = idx_t[ch, pl.ds(seg * _L, _L)]
            vc = idx_c[ch, pl.ds(seg * _L, _L)]
            rt = seg * _L + lane
            ht = (vt % 2) * _EMBED_DIM
            hc = (vc % 2) * _EMBED_DIM
            acc = jnp.zeros((_L,), jnp.float32)
            for d in range(_EMBED_DIM):
                t = plsc.load_gather(rows_t, [rt, ht + d])
                c = plsc.load_gather(rows_c, [rt, hc + d])
                acc = acc + t * c
            e = jnp.exp(-jnp.abs(acc))
            r_ = 1.0 / (1.0 + e)
            sig = jnp.where(acc >= 0, r_, e * r_)
            out_v[pl.ds(ch * _CHUNK + seg * _L, _L)] = sig
        return carry

    lax.fori_loop(0, _NCH, chunk, 0)

    pltpu.sync_copy(out_v, out_hbm.at[pl.ds(wid * _BPW, _BPW)])


@jax.jit
def _run(target_i, context_j, shared_embedding):
    mesh = plsc.VectorSubcoreMesh(core_axis_name="c", subcore_axis_name="s")
    tgt2d = target_i.reshape(_NW * _IDXROWS, 128)
    ctx2d = context_j.reshape(_NW * _IDXROWS, 128)
    tabp = shared_embedding.reshape(_VOCAB // 2, 2 * _EMBED_DIM)
    kern = functools.partial(
        pl.kernel,
        out_type=jax.ShapeDtypeStruct((_BATCH,), jnp.float32),
        mesh=mesh,
        scratch_types=[
            pltpu.VMEM((_IDXROWS, 128), jnp.int32),
            pltpu.VMEM((_IDXROWS, 128), jnp.int32),
            pltpu.VMEM((1, _CHUNK), jnp.int32),
            pltpu.VMEM((1, _CHUNK), jnp.int32),
            pltpu.VMEM((_CHUNK, 2 * _EMBED_DIM), jnp.float32),
            pltpu.VMEM((_CHUNK, 2 * _EMBED_DIM), jnp.float32),
            pltpu.VMEM((_BPW,), jnp.float32),
            pltpu.SemaphoreType.DMA,
        ],
        compiler_params=pltpu.CompilerParams(needs_layout_passes=False),
    )(_sc_body)
    return kern(tgt2d, ctx2d, tabp)


def kernel(target_i, context_j, shared_embedding):
    return _run(target_i.astype(jnp.int32), context_j.astype(jnp.int32),
                shared_embedding)
